# baseline (speedup 1.0000x reference)
import functools

import jax
import jax.numpy as jnp
from jax import lax
from jax.experimental import pallas as pl
from jax.experimental.pallas import tpu as pltpu
from jax.experimental.pallas import tpu_sc as plsc

VOCAB = 1000000
EMBED = 64
PAD_IDX = 0

NC, NS, L = 2, 16, 16
NW = NC * NS
B_TOTAL = 4096 * 50
B_PER_W = B_TOTAL // NW
G = 128
ROWS_PER_W = B_PER_W // G
CHUNK_G = 5
CHUNK = CHUNK_G * G
NCHUNK = B_PER_W // CHUNK

TC_FULL = 7808
TC_PER_W = TC_FULL // NW
NBUF = 4
TAIL_V = VOCAB - TC_FULL * G
TAIL_W = TAIL_V * EMBED


def _in_copy(tabT_hbm, blk, lane0, fire, sem):
    mk = pltpu.async_copy if fire else pltpu.make_async_copy
    return mk(tabT_hbm.at[:, pl.ds(lane0, G)], blk, sem)


def _transpose_block(blk, ob, nlanes, vi, vi64, rots):
    @plsc.parallel_loop(0, (nlanes // L) * (EMBED // L), unroll=4)
    def _v_body(m):
        g = m >> 2
        j0 = (m & 3) << 4
        iv = vi + g * L
        basev = vi64 + g * (L * EMBED)
        for d in range(L):
            ijd = rots[d] + j0
            vec = plsc.load_gather(blk, [ijd, iv])
            plsc.store_scatter(ob, [basev + ijd], vec)


def _detile_kernel(tabT_hbm, tail_hbm, out_hbm, blks, obs, isems, osems):
    wid = lax.axis_index("s") * NC + lax.axis_index("c")
    tc_base = wid * TC_PER_W
    vi = jnp.arange(L, dtype=jnp.int32)
    vi64 = vi * EMBED
    rots = [jnp.bitwise_and(vi + d, L - 1) for d in range(L)]
    ob_words = G * EMBED

    def lane0_of(c):
        return (tc_base + c) * G

    def out_off(c):
        return (tc_base + c) * (G * EMBED)

    for k in range(NBUF - 1):
        _in_copy(tabT_hbm, blks[k], lane0_of(k), True, isems[k])

    def body(i, carry):
        for k in range(NBUF):
            c = NBUF * i + k
            _in_copy(tabT_hbm, blks[k], lane0_of(c), False, isems[k]).wait()

            @pl.when(i > 0)
            def _drain_prev_store(k=k):
                pltpu.make_async_copy(
                    obs[k], out_hbm.at[pl.ds(0, ob_words)], osems[k]).wait()

            _transpose_block(blks[k], obs[k], G, vi, vi64, rots)

            @pl.when(jnp.logical_and(wid == 0, c == 0))
            def _zero_pad_row(k=k):
                zeros = jnp.zeros((L,), jnp.float32)
                for cc in range(EMBED // L):
                    obs[k][pl.ds(PAD_IDX * EMBED + cc * L, L)] = zeros

            pltpu.async_copy(
                obs[k], out_hbm.at[pl.ds(out_off(c), ob_words)], osems[k])

            nxt = (k + NBUF - 1) % NBUF

            @pl.when(c + NBUF - 1 < TC_PER_W)
            def _prefetch(k=k, nxt=nxt, c=c):
                _in_copy(tabT_hbm, blks[nxt], lane0_of(c + NBUF - 1), True,
                         isems[nxt])

        return carry

    lax.fori_loop(0, TC_PER_W // NBUF, body, 0)
    for k in range(NBUF):
        pltpu.make_async_copy(
            obs[k], out_hbm.at[pl.ds(0, ob_words)], osems[k]).wait()

    base = TC_FULL * G * EMBED

    @pl.when(wid < 4)
    def _tail_full():
        off = wid * (G * EMBED)
        pltpu.sync_copy(tail_hbm.at[pl.ds(off, G * EMBED)],
                        obs[0].at[pl.ds(0, G * EMBED)])
        pltpu.sync_copy(obs[0].at[pl.ds(0, G * EMBED)],
                        out_hbm.at[pl.ds(base + off, G * EMBED)])

    @pl.when(wid == 4)
    def _tail_half():
        off = 4 * (G * EMBED)
        n = TAIL_W - off
        pltpu.sync_copy(tail_hbm.at[pl.ds(off, n)], obs[0].at[pl.ds(0, n)])
        pltpu.sync_copy(obs[0].at[pl.ds(0, n)],
                        out_hbm.at[pl.ds(base + off, n)])


def _gather_kernel(idx_hbm, table_hbm, out_hbm, idx_v, rows0, rows1,
                   gsem0, gsem1, osem0, osem1):
    wid = lax.axis_index("s") * NC + lax.axis_index("c")
    wbase = wid * B_PER_W

    pltpu.sync_copy(idx_hbm.at[pl.ds(wid * ROWS_PER_W, ROWS_PER_W)], idx_v)

    rows = (rows0, rows1)
    gsems = (gsem0, gsem1)
    osems = (osem0, osem1)

    def fire(c, b):
        for j in range(CHUNK_G):
            pltpu.async_copy(table_hbm.at[idx_v.at[c * CHUNK_G + j]],
                             rows[b].at[pl.ds(j * G, G)], gsems[b])

    fire(0, 0)
    for c in range(NCHUNK):
        b = c % 2
        for j in range(CHUNK_G):
            pltpu.make_async_copy(table_hbm.at[idx_v.at[c * CHUNK_G + j]],
                                  rows[b].at[pl.ds(j * G, G)],
                                  gsems[b]).wait()
        if c + 1 < NCHUNK:
            if c >= 1:
                pltpu.make_async_copy(
                    rows[1 - b],
                    out_hbm.at[pl.ds(wbase + (c - 1) * CHUNK, CHUNK)],
                    osems[1 - b]).wait()
            fire(c + 1, 1 - b)
        pltpu.async_copy(rows[b],
                         out_hbm.at[pl.ds(wbase + c * CHUNK, CHUNK)],
                         osems[b])
    for b in range(2):
        pltpu.make_async_copy(rows[b], out_hbm.at[pl.ds(wbase, CHUNK)],
                              osems[b]).wait()


@jax.jit
def kernel(src_indices, src_table):
    idx = src_indices.reshape(NW * ROWS_PER_W, G).astype(jnp.int32)
    mesh = plsc.VectorSubcoreMesh(core_axis_name="c", subcore_axis_name="s")
    lin = pl.kernel(
        _detile_kernel,
        out_type=jax.ShapeDtypeStruct((VOCAB * EMBED,), jnp.float32),
        mesh=mesh,
        scratch_types=[
            [pltpu.VMEM((EMBED, G), jnp.float32) for _ in range(NBUF)],
            [pltpu.VMEM((G * EMBED,), jnp.float32) for _ in range(NBUF)],
            [pltpu.SemaphoreType.DMA for _ in range(NBUF)],
            [pltpu.SemaphoreType.DMA for _ in range(NBUF)],
        ],
        compiler_params=pltpu.CompilerParams(use_tc_tiling_on_sc=True,
                                             needs_layout_passes=False),
    )(src_table.T, src_table[TC_FULL * G:].reshape(TAIL_W))
    out = pl.kernel(
        _gather_kernel,
        out_type=jax.ShapeDtypeStruct((B_TOTAL, EMBED), jnp.float32),
        mesh=mesh,
        scratch_types=[
            pltpu.VMEM((ROWS_PER_W, G), jnp.int32),
            pltpu.VMEM((CHUNK, EMBED), jnp.float32),
            pltpu.VMEM((CHUNK, EMBED), jnp.float32),
            pltpu.SemaphoreType.DMA,
            pltpu.SemaphoreType.DMA,
            pltpu.SemaphoreType.DMA,
            pltpu.SemaphoreType.DMA,
        ],
        compiler_params=pltpu.CompilerParams(use_tc_tiling_on_sc=False),
    )(idx, lin.reshape(VOCAB, EMBED))
    return out.reshape(4096, 50, EMBED)


# speedup vs baseline: 2.2195x; 2.2195x over previous
import functools

import jax
import jax.numpy as jnp
from jax import lax
from jax.experimental import pallas as pl
from jax.experimental.pallas import tpu as pltpu
from jax.experimental.pallas import tpu_sc as plsc

VOCAB = 1000000
EMBED = 64
PAD_IDX = 0

NC, NS, L = 2, 16, 16
NW = NC * NS
B_TOTAL = 4096 * 50
B_PER_W = B_TOTAL // NW
G = 128
ROWS_PER_W = B_PER_W // G
CHUNK_G = 5
CHUNK = CHUNK_G * G
NCHUNK = B_PER_W // CHUNK

TC_FULL = 7808
TC_PER_W = TC_FULL // NW
NBUF = 4
TAIL_V = VOCAB - TC_FULL * G
TAIL_W = TAIL_V * EMBED


def _in_copy(tabT_hbm, blk, lane0, fire, sem):
    mk = pltpu.async_copy if fire else pltpu.make_async_copy
    return mk(tabT_hbm.at[:, pl.ds(lane0, G)], blk, sem)


def _transpose_block(blk, ob, nlanes, vi, vi64, rots):
    @plsc.parallel_loop(0, (nlanes // L) * (EMBED // L), unroll=2)
    def _v_body(m):
        g = m >> 2
        j0 = (m & 3) << 4
        iv = vi + g * L
        basev = vi64 + g * (L * EMBED)
        for d in range(L):
            ijd = rots[d] + j0
            vec = plsc.load_gather(blk, [ijd, iv])
            plsc.store_scatter(ob, [basev + ijd], vec)


def _detile_kernel(tabT_hbm, tail_hbm, out_hbm, blks, obs, isems, osems):
    wid = lax.axis_index("s") * NC + lax.axis_index("c")
    tc_base = wid * TC_PER_W
    vi = jnp.arange(L, dtype=jnp.int32)
    vi64 = vi * EMBED
    rots = [jnp.bitwise_and(vi + d, L - 1) for d in range(L)]
    ob_words = G * EMBED

    def lane0_of(c):
        return (tc_base + c) * G

    def out_off(c):
        return (tc_base + c) * (G * EMBED)

    for k in range(NBUF - 1):
        _in_copy(tabT_hbm, blks[k], lane0_of(k), True, isems[k])

    def body(i, carry):
        for k in range(NBUF):
            c = NBUF * i + k
            _in_copy(tabT_hbm, blks[k], lane0_of(c), False, isems[k]).wait()

            @pl.when(i > 0)
            def _drain_prev_store(k=k):
                pltpu.make_async_copy(
                    obs[k], out_hbm.at[pl.ds(0, ob_words)], osems[k]).wait()

            _transpose_block(blks[k], obs[k], G, vi, vi64, rots)

            @pl.when(jnp.logical_and(wid == 0, c == 0))
            def _zero_pad_row(k=k):
                zeros = jnp.zeros((L,), jnp.float32)
                for cc in range(EMBED // L):
                    obs[k][pl.ds(PAD_IDX * EMBED + cc * L, L)] = zeros

            pltpu.async_copy(
                obs[k], out_hbm.at[pl.ds(out_off(c), ob_words)], osems[k])

            nxt = (k + NBUF - 1) % NBUF

            @pl.when(c + NBUF - 1 < TC_PER_W)
            def _prefetch(k=k, nxt=nxt, c=c):
                _in_copy(tabT_hbm, blks[nxt], lane0_of(c + NBUF - 1), True,
                         isems[nxt])

        return carry

    lax.fori_loop(0, TC_PER_W // NBUF, body, 0)
    for k in range(NBUF):
        pltpu.make_async_copy(
            obs[k], out_hbm.at[pl.ds(0, ob_words)], osems[k]).wait()

    base = TC_FULL * G * EMBED

    @pl.when(wid < 4)
    def _tail_full():
        off = wid * (G * EMBED)
        pltpu.sync_copy(tail_hbm.at[pl.ds(off, G * EMBED)],
                        obs[0].at[pl.ds(0, G * EMBED)])
        pltpu.sync_copy(obs[0].at[pl.ds(0, G * EMBED)],
                        out_hbm.at[pl.ds(base + off, G * EMBED)])

    @pl.when(wid == 4)
    def _tail_half():
        off = 4 * (G * EMBED)
        n = TAIL_W - off
        pltpu.sync_copy(tail_hbm.at[pl.ds(off, n)], obs[0].at[pl.ds(0, n)])
        pltpu.sync_copy(obs[0].at[pl.ds(0, n)],
                        out_hbm.at[pl.ds(base + off, n)])


def _gather_kernel(idx_hbm, table_hbm, out_hbm, idx_v, rows0, rows1,
                   gsem0, gsem1, osem0, osem1):
    wid = lax.axis_index("s") * NC + lax.axis_index("c")
    wbase = wid * B_PER_W

    pltpu.sync_copy(idx_hbm.at[pl.ds(wid * ROWS_PER_W, ROWS_PER_W)], idx_v)

    rows = (rows0, rows1)
    gsems = (gsem0, gsem1)
    osems = (osem0, osem1)

    def fire(c, b):
        for j in range(CHUNK_G):
            pltpu.async_copy(table_hbm.at[idx_v.at[c * CHUNK_G + j]],
                             rows[b].at[pl.ds(j * G, G)], gsems[b])

    fire(0, 0)
    for c in range(NCHUNK):
        b = c % 2
        for j in range(CHUNK_G):
            pltpu.make_async_copy(table_hbm.at[idx_v.at[c * CHUNK_G + j]],
                                  rows[b].at[pl.ds(j * G, G)],
                                  gsems[b]).wait()
        if c + 1 < NCHUNK:
            if c >= 1:
                pltpu.make_async_copy(
                    rows[1 - b],
                    out_hbm.at[pl.ds(wbase + (c - 1) * CHUNK, CHUNK)],
                    osems[1 - b]).wait()
            fire(c + 1, 1 - b)
        pltpu.async_copy(rows[b],
                         out_hbm.at[pl.ds(wbase + c * CHUNK, CHUNK)],
                         osems[b])
    for b in range(2):
        pltpu.make_async_copy(rows[b], out_hbm.at[pl.ds(wbase, CHUNK)],
                              osems[b]).wait()


@jax.jit
def kernel(src_indices, src_table):
    idx = src_indices.reshape(NW * ROWS_PER_W, G).astype(jnp.int32)
    mesh = plsc.VectorSubcoreMesh(core_axis_name="c", subcore_axis_name="s")
    lin = pl.kernel(
        _detile_kernel,
        out_type=jax.ShapeDtypeStruct((VOCAB * EMBED,), jnp.float32),
        mesh=mesh,
        scratch_types=[
            [pltpu.VMEM((EMBED, G), jnp.float32) for _ in range(NBUF)],
            [pltpu.VMEM((G * EMBED,), jnp.float32) for _ in range(NBUF)],
            [pltpu.SemaphoreType.DMA for _ in range(NBUF)],
            [pltpu.SemaphoreType.DMA for _ in range(NBUF)],
        ],
        compiler_params=pltpu.CompilerParams(use_tc_tiling_on_sc=True,
                                             needs_layout_passes=False),
    )(src_table.T, src_table[TC_FULL * G:].reshape(TAIL_W))
    out = pl.kernel(
        _gather_kernel,
        out_type=jax.ShapeDtypeStruct((B_TOTAL, EMBED), jnp.float32),
        mesh=mesh,
        scratch_types=[
            pltpu.VMEM((ROWS_PER_W, G), jnp.int32),
            pltpu.VMEM((CHUNK, EMBED), jnp.float32),
            pltpu.VMEM((CHUNK, EMBED), jnp.float32),
            pltpu.SemaphoreType.DMA,
            pltpu.SemaphoreType.DMA,
            pltpu.SemaphoreType.DMA,
            pltpu.SemaphoreType.DMA,
        ],
        compiler_params=pltpu.CompilerParams(use_tc_tiling_on_sc=False),
    )(idx, lin.reshape(VOCAB, EMBED))
    return out.reshape(4096, 50, EMBED)
